# Initial kernel scaffold; baseline (speedup 1.0000x reference)
import jax
import jax.numpy as jnp
from jax.experimental import pallas as pl


def kernel(x, edge_index, enc12_fcW, enc12_fcb, enc12_cW, enc12_cb, enc23_fcW, enc23_fcb, enc23_cW, enc23_cb, proj_fcW, proj_fcb, proj_cW, proj_cb, mlpW, mlpb, fc1W, fc1b, att_w, att_u, clf_fc1W, clf_fc1b, clf_c1W, clf_c1b, clf_fc2W, clf_fc2b, clf_c2W, clf_c2b):
    raise NotImplementedError("write your pallas kernel here")


# speedup vs baseline: 8.1697x; 8.1697x over previous
import functools

import jax
import jax.numpy as jnp
from jax import lax
from jax.experimental import pallas as pl
from jax.experimental.pallas import tpu as pltpu
from jax.experimental.pallas import tpu_sc as plsc

N_NODES = 50000
SPAN = 3200
N_PAD = 16 * SPAN
E_PAD = 819200
LANES = 128
G_TOTAL = E_PAD // LANES
NC, NS = 2, 16

_mesh = lambda: plsc.VectorSubcoreMesh(core_axis_name="c", subcore_axis_name="s")
_SC_PARAMS = pltpu.CompilerParams(use_tc_tiling_on_sc=False)


def _make_wide_prop(Wc):
    GPT = G_TOTAL // NS
    CH = 5
    NB = GPT // CH

    @functools.partial(
        pl.kernel,
        out_type=jax.ShapeDtypeStruct((NC, N_PAD, Wc), jnp.float32),
        mesh=_mesh(),
        scratch_types=[
            pltpu.VMEM((CH, LANES), jnp.int32),
            pltpu.VMEM((CH, LANES), jnp.int32),
            pltpu.VMEM((CH, LANES, Wc), jnp.float32),
            pltpu.VMEM_SHARED((N_PAD, Wc), jnp.float32),
            pltpu.SemaphoreType.DMA,
        ],
        compiler_params=_SC_PARAMS,
    )
    def prop(tab, row2d, col2d, zrows, out, row_v, col_v, rows_v, acc, sem):
        c = lax.axis_index("c")
        s = lax.axis_index("s")
        pltpu.sync_copy(zrows, acc.at[pl.ds(s * SPAN, SPAN), :])
        plsc.subcore_barrier()

        def block(b, _):
            base = s * GPT + b * CH
            pltpu.sync_copy(row2d.at[pl.ds(base, CH), :], row_v)
            pltpu.sync_copy(col2d.at[pl.ds(base, CH), :], col_v)
            for g in range(CH):
                pltpu.async_copy(tab.at[c].at[row_v.at[g]], rows_v.at[g], sem)
            for g in range(CH):
                pltpu.make_async_copy(tab.at[c].at[row_v.at[g]], rows_v.at[g],
                                      sem).wait()
            for g in range(CH):
                pltpu.sync_copy(rows_v.at[g], acc.at[col_v.at[g]], add=True)
            return 0

        lax.fori_loop(0, NB, block, 0)
        plsc.subcore_barrier()
        pltpu.sync_copy(acc.at[pl.ds(s * SPAN, SPAN), :],
                        out.at[c].at[pl.ds(s * SPAN, SPAN), :])

    return prop


def _make_scalar_prop(gather):
    GPT = G_TOTAL // (NC * NS)
    CH = 8
    NB = GPT // CH
    scratch = [
        pltpu.VMEM((CH, LANES), jnp.int32),
        pltpu.VMEM((CH, LANES), jnp.float32),
        pltpu.VMEM_SHARED((N_PAD,), jnp.float32),
        pltpu.SemaphoreType.DMA,
    ]
    if gather:
        scratch.insert(0, pltpu.VMEM((CH, LANES), jnp.int32))

    @functools.partial(
        pl.kernel,
        out_type=jax.ShapeDtypeStruct((NC, N_PAD), jnp.float32),
        mesh=_mesh(),
        scratch_types=scratch,
        compiler_params=_SC_PARAMS,
    )
    def prop(*refs):
        if gather:
            table, row2d, col2d, zrow, vsrc, out, row_v, col_v, val_v, acc, sem = refs
        else:
            row2d, col2d, zrow, vsrc, out, col_v, val_v, acc, sem = refs
        c = lax.axis_index("c")
        s = lax.axis_index("s")
        pltpu.sync_copy(zrow, acc.at[pl.ds(s * SPAN, SPAN)])
        if not gather:
            pltpu.sync_copy(vsrc, val_v)
        plsc.subcore_barrier()

        def block(b, _):
            base = (c * NS + s) * GPT + b * CH
            pltpu.sync_copy(col2d.at[pl.ds(base, CH), :], col_v)
            if gather:
                pltpu.sync_copy(row2d.at[pl.ds(base, CH), :], row_v)
                for g in range(CH):
                    pltpu.async_copy(table.at[row_v.at[g]], val_v.at[g], sem)
                for g in range(CH):
                    pltpu.make_async_copy(table.at[row_v.at[g]], val_v.at[g],
                                          sem).wait()
                for g in range(CH):
                    pltpu.sync_copy(val_v.at[g], acc.at[col_v.at[g]], add=True)
            else:
                for g in range(CH):
                    pltpu.sync_copy(val_v.at[g], acc.at[col_v.at[g]], add=True)
            return 0

        lax.fori_loop(0, NB, block, 0)
        plsc.subcore_barrier()
        pltpu.sync_copy(acc.at[pl.ds(s * SPAN, SPAN)],
                        out.at[c].at[pl.ds(s * SPAN, SPAN)])

    return prop


_prop32 = _make_wide_prop(32)
_deg_kernel = _make_scalar_prop(gather=False)
_p3_kernel = _make_scalar_prop(gather=True)


BN = 1000
GRID = N_NODES // BN


def _spec_for(shape):
    if len(shape) == 3 and shape[1] in (N_NODES, N_PAD):
        return pl.BlockSpec((shape[0], BN, shape[2]), lambda i: (0, i, 0))
    if len(shape) == 2 and shape[0] in (N_NODES, N_PAD):
        return pl.BlockSpec((BN, shape[1]), lambda i: (i, 0))
    nd = len(shape)
    return pl.BlockSpec(shape, lambda i: (0,) * nd)


def _tc_call(body, in_arrays, out_shapes):
    in_specs = [_spec_for(a.shape) for a in in_arrays]
    out_specs = [_spec_for(s) for s in out_shapes]
    return pl.pallas_call(
        body,
        grid=(GRID,),
        in_specs=in_specs,
        out_specs=out_specs if len(out_specs) > 1 else out_specs[0],
        out_shape=([jax.ShapeDtypeStruct(s, jnp.float32) for s in out_shapes]
                   if len(out_shapes) > 1
                   else jax.ShapeDtypeStruct(out_shapes[0], jnp.float32)),
    )(*in_arrays)


def _dot(a, b):
    return jnp.dot(a, b, preferred_element_type=jnp.float32)


def kernel(x, edge_index, enc12_fcW, enc12_fcb, enc12_cW, enc12_cb,
           enc23_fcW, enc23_fcb, enc23_cW, enc23_cb, proj_fcW, proj_fcb,
           proj_cW, proj_cb, mlpW, mlpb, fc1W, fc1b, att_w, att_u,
           clf_fc1W, clf_fc1b, clf_c1W, clf_c1b, clf_fc2W, clf_fc2b,
           clf_c2W, clf_c2b):
    f32 = jnp.float32
    row = edge_index[0]
    col = edge_index[1]
    pad = E_PAD - row.shape[0]
    row2d = jnp.concatenate([row, jnp.zeros((pad,), jnp.int32)]).reshape(G_TOTAL, LANES)
    trash = N_NODES + jnp.arange(pad, dtype=jnp.int32) % (N_PAD - N_NODES)
    col2d = jnp.concatenate([col, trash]).reshape(G_TOTAL, LANES)
    z1 = jnp.zeros((SPAN,), f32)
    z32 = jnp.zeros((SPAN, 32), f32)
    ones_src = jnp.ones((8, LANES), f32)

    degp = _deg_kernel(row2d, col2d, z1, ones_src)
    d0 = degp[0, :N_NODES].reshape(N_NODES, 1)
    d1 = degp[1, :N_NODES].reshape(N_NODES, 1)

    def pre_body(x_r, d0_r, d1_r, c12_r, c23_r, dinv_r, t1_r, t2_r, t3_r, t4_r):
        deg = d0_r[...] + d1_r[...]
        dinv = jnp.where(deg > 0, lax.rsqrt(jnp.maximum(deg, 1.0)), 0.0)
        dinv_r[...] = dinv
        xb = x_r[...]
        f1 = xb[:, 16:32]
        f2 = xb[:, 32:48]
        f3 = xb[:, 0:16]
        for t_r, f, w in ((t1_r, f1, c12_r), (t2_r, f2, c12_r),
                          (t3_r, f2, c23_r), (t4_r, f3, c23_r)):
            t = _dot(f, w[...]) * dinv
            t_r[0] = t[:, :32]
            t_r[1] = t[:, 32:]

    dinv, T1, T2, T3, T4 = _tc_call(
        pre_body, [x, d0, d1, enc12_cW, enc23_cW],
        [(N_NODES, 1)] + [(NC, N_NODES, 32)] * 4)

    G1 = _prop32(T1, row2d, col2d, z32)
    G2 = _prop32(T2, row2d, col2d, z32)
    G3 = _prop32(T3, row2d, col2d, z32)
    G4 = _prop32(T4, row2d, col2d, z32)

    def enc_body(x_r, g1_r, g2_r, g3_r, g4_r, dinv_r, f12_r, f23_r, bf12_r,
                 bf23_r, bc12_r, bc23_r, mlp_r, bm_r, f1a_r, f1b_r, b1_r,
                 aw_r, au_r, pfW_r, pfb_r, pcW_r, t5a_r, t5b_r, fcp_r):
        dinv = dinv_r[...]
        xb = x_r[...]
        f1 = xb[:, 16:32]
        f2 = xb[:, 32:48]
        f3 = xb[:, 0:16]

        def enc(f, fw, fb, g_r, cb):
            g = jnp.concatenate([g_r[0], g_r[1]], axis=1) * dinv
            return jnp.maximum(_dot(f, fw) + fb, 0.0) + g + cb

        e1 = enc(f1, f12_r[...], bf12_r[...], g1_r, bc12_r[...])
        e21 = enc(f2, f12_r[...], bf12_r[...], g2_r, bc12_r[...])
        e23 = enc(f2, f23_r[...], bf23_r[...], g3_r, bc23_r[...])
        e3 = enc(f3, f23_r[...], bf23_r[...], g4_r, bc23_r[...])
        q21 = jnp.maximum(_dot(e21, mlp_r[...]) + bm_r[...], 0.0)
        q23 = jnp.maximum(_dot(e23, mlp_r[...]) + bm_r[...], 0.0)
        e2 = _dot(q21, f1a_r[...]) + _dot(q23, f1b_r[...]) + b1_r[...]
        aw = aw_r[...]
        au = au_r[...]
        vu1 = _dot(jnp.tanh(_dot(e1, aw)), au)
        vu2 = _dot(jnp.tanh(_dot(e2, aw)), au)
        vu3 = _dot(jnp.tanh(_dot(e3, aw)), au)
        m = jnp.maximum(jnp.maximum(vu1, vu2), vu3)
        a1 = jnp.exp(vu1 - m)
        a2 = jnp.exp(vu2 - m)
        a3 = jnp.exp(vu3 - m)
        inv = 1.0 / (a1 + a2 + a3)
        comb = (a1 * e1 + a2 * e2 + a3 * e3) * inv
        fcp_r[...] = jnp.maximum(_dot(comb, pfW_r[...]) + pfb_r[...], 0.0)
        t5 = _dot(comb, pcW_r[...]) * dinv
        t5a_r[0] = t5[:, 0:32]
        t5a_r[1] = t5[:, 32:64]
        t5b_r[0] = t5[:, 64:96]
        t5b_r[1] = jnp.concatenate(
            [t5[:, 96:100], jnp.zeros((t5.shape[0], 28), t5.dtype)], axis=1)

    T5a, T5b, fcp = _tc_call(
        enc_body,
        [x, G1, G2, G3, G4, dinv, enc12_fcW, enc23_fcW,
         enc12_fcb.reshape(1, 64), enc23_fcb.reshape(1, 64),
         enc12_cb.reshape(1, 64), enc23_cb.reshape(1, 64), mlpW,
         mlpb.reshape(1, 64), fc1W[:64], fc1W[64:], fc1b.reshape(1, 64),
         att_w, att_u, proj_fcW, proj_fcb.reshape(1, 100), proj_cW],
        [(NC, N_NODES, 32), (NC, N_NODES, 32), (N_NODES, 100)])

    G5a = _prop32(T5a, row2d, col2d, z32)
    G5b = _prop32(T5b, row2d, col2d, z32)

    def mid_body(g5a_r, g5b_r, dinv_r, fcp_r, pcb_r, c1W_r, c1b_r, k1W_r,
                 t6_r, h0_r):
        dinv = dinv_r[...]
        g5 = jnp.concatenate([g5a_r[0], g5a_r[1], g5b_r[0],
                              g5b_r[1][:, 0:4]], axis=1) * dinv
        comb2 = fcp_r[...] + g5 + pcb_r[...]
        h0_r[...] = jnp.maximum(_dot(comb2, c1W_r[...]) + c1b_r[...], 0.0)
        t6 = _dot(comb2, k1W_r[...]) * dinv
        t6_r[0] = t6[:, :32]
        t6_r[1] = t6[:, 32:]

    T6, h0 = _tc_call(
        mid_body,
        [G5a, G5b, dinv, fcp, proj_cb.reshape(1, 100), clf_fc1W,
         clf_fc1b.reshape(1, 64), clf_c1W],
        [(NC, N_NODES, 32), (N_NODES, 64)])

    G6 = _prop32(T6, row2d, col2d, z32)

    def clf_body(g6_r, dinv_r, h0_r, c1b_r, f2W_r, c2W_r, ob_r, u_r, o0_r):
        dinv = dinv_r[...]
        g6 = jnp.concatenate([g6_r[0], g6_r[1]], axis=1) * dinv
        h = h0_r[...] + jnp.maximum(g6 + c1b_r[...], 0.0)
        o0_r[...] = _dot(h, f2W_r[...]) + ob_r[...]
        u_r[...] = _dot(h, c2W_r[...]) * dinv

    u, o0p = _tc_call(
        clf_body,
        [G6, dinv, h0, clf_c1b.reshape(1, 64), clf_fc2W, clf_c2W,
         (clf_fc2b + clf_c2b).reshape(1, 1)],
        [(N_NODES, 1), (N_NODES, 1)])

    up = jnp.concatenate([u.reshape(N_NODES), jnp.zeros((N_PAD - N_NODES,), f32)])
    p3s = _p3_kernel(up, row2d, col2d, z1, ones_src)

    def fin_body(p0_r, p1_r, dinv_r, o0_r, out_r):
        out_r[...] = (p0_r[...] + p1_r[...]) * dinv_r[...] + o0_r[...]

    out = _tc_call(
        fin_body,
        [p3s[0, :N_NODES].reshape(N_NODES, 1), p3s[1, :N_NODES].reshape(N_NODES, 1),
         dinv, o0p],
        [(N_NODES, 1)])
    return out
